# baseline (speedup 1.0000x reference)
import functools

import jax
import jax.numpy as jnp
from jax import lax
from jax.experimental import pallas as pl
from jax.experimental.pallas import tpu as pltpu
from jax.experimental.pallas import tpu_sc as plsc

NC = 2
NS = 16
L = 16
NW = NC * NS
B = 16384
D = 128
BPW = B // NW
C = 64
NCH = BPW // C
NSL = D // L
NBUF = 4


def _start_chunk(ent_hbm, rel_hbm, hi_v, ri_v, ti_v, h_rows, r_rows, t_rows,
                 sems, j, b):
    sl = pl.ds(j * C, C)
    pltpu.async_copy(ent_hbm.at[hi_v.at[sl]], h_rows.at[b], sems.at[b, 0])
    pltpu.async_copy(rel_hbm.at[ri_v.at[sl]], r_rows.at[b], sems.at[b, 1])
    pltpu.async_copy(ent_hbm.at[ti_v.at[sl]], t_rows.at[b], sems.at[b, 2])


def _wait_chunk(ent_hbm, rel_hbm, hi_v, ri_v, ti_v, h_rows, r_rows, t_rows,
                sems, j, b):
    sl = pl.ds(j * C, C)
    pltpu.make_async_copy(ent_hbm.at[hi_v.at[sl]], h_rows.at[b],
                          sems.at[b, 0]).wait()
    pltpu.make_async_copy(rel_hbm.at[ri_v.at[sl]], r_rows.at[b],
                          sems.at[b, 1]).wait()
    pltpu.make_async_copy(ent_hbm.at[ti_v.at[sl]], t_rows.at[b],
                          sems.at[b, 2]).wait()


def _body(heads_hbm, rels_hbm, tails_hbm, ent_hbm, rel_hbm, out_hbm,
          hi_v, ri_v, ti_v, h_rows, r_rows, t_rows, part_v, out_v, sems):
    wid = lax.axis_index("s") * NC + lax.axis_index("c")
    base = wid * BPW

    ci = pltpu.async_copy(heads_hbm.at[pl.ds(base, BPW)], hi_v, sems.at[0, 0])
    cr = pltpu.async_copy(rels_hbm.at[pl.ds(base, BPW)], ri_v, sems.at[0, 1])
    ct = pltpu.async_copy(tails_hbm.at[pl.ds(base, BPW)], ti_v, sems.at[0, 2])
    ci.wait()
    cr.wait()
    ct.wait()

    lane = lax.iota(jnp.int32, L)
    lane_l = lane * L

    args = (ent_hbm, rel_hbm, hi_v, ri_v, ti_v, h_rows, r_rows, t_rows, sems)
    for b in range(NBUF):
        _start_chunk(*args, b, b)

    def compute_chunk(j, b):
        hb, rb, tb, pv = h_rows.at[b], r_rows.at[b], t_rows.at[b], part_v

        def row_body(r, carry):
            ps = [hb[r, pl.ds(k * L, L)] * rb[r, pl.ds(k * L, L)]
                  * tb[r, pl.ds(k * L, L)] for k in range(NSL)]
            while len(ps) > 1:
                ps = [ps[i] + ps[i + 1] for i in range(0, len(ps), 2)]
            pv[pl.ds(r * L, L)] = ps[0]
            return carry

        lax.fori_loop(0, C, row_body, 0, unroll=8)

        def group_body(g, carry):
            idx0 = g * (L * L) + lane_l
            acc = plsc.load_gather(part_v, [idx0])
            for k in range(1, L):
                acc += plsc.load_gather(part_v, [idx0 + k])
            out_v[pl.ds(j * C + g * L, L)] = acc
            return carry

        lax.fori_loop(0, C // L, group_body, 0)

    def outer_body(i, carry):
        for b in range(NBUF):
            j = i * NBUF + b
            _wait_chunk(*args, j, b)
            compute_chunk(j, b)

            @pl.when(i < NCH // NBUF - 1)
            def _():
                _start_chunk(*args, j + NBUF, b)
        return carry

    lax.fori_loop(0, NCH // NBUF, outer_body, 0)

    pltpu.sync_copy(out_v, out_hbm.at[pl.ds(base, BPW)])


@jax.jit
def kernel(heads, relations, tails, entity_emb, relation_emb):
    f = pl.kernel(
        _body,
        out_type=jax.ShapeDtypeStruct((B,), jnp.float32),
        mesh=plsc.VectorSubcoreMesh(core_axis_name="c", subcore_axis_name="s",
                                    num_cores=NC, num_subcores=NS),
        compiler_params=pltpu.CompilerParams(needs_layout_passes=False),
        scratch_types=[
            pltpu.VMEM((BPW,), jnp.int32),
            pltpu.VMEM((BPW,), jnp.int32),
            pltpu.VMEM((BPW,), jnp.int32),
            pltpu.VMEM((NBUF, C, D), jnp.float32),
            pltpu.VMEM((NBUF, C, D), jnp.float32),
            pltpu.VMEM((NBUF, C, D), jnp.float32),
            pltpu.VMEM((C * L,), jnp.float32),
            pltpu.VMEM((BPW,), jnp.float32),
            pltpu.SemaphoreType.DMA((NBUF, 3)),
        ],
    )
    out = f(heads.astype(jnp.int32), relations.astype(jnp.int32),
            tails.astype(jnp.int32), entity_emb, relation_emb)
    return out.reshape(B, 1)


# speedup vs baseline: 1.0887x; 1.0887x over previous
import functools

import jax
import jax.numpy as jnp
from jax import lax
from jax.experimental import pallas as pl
from jax.experimental.pallas import tpu as pltpu
from jax.experimental.pallas import tpu_sc as plsc

NC = 2
NS = 16
L = 16
NW = NC * NS
B = 16384
D = 128
BPW = B // NW
C = 64
NCH = BPW // C
NSL = D // L
NBUF = 4


def _start_chunk(ent_hbm, rel_hbm, hi_v, ri_v, ti_v, h_rows, r_rows, t_rows,
                 sems, j, b):
    sl = pl.ds(j * C, C)
    pltpu.async_copy(ent_hbm.at[hi_v.at[sl]], h_rows.at[b], sems.at[b, 0])
    pltpu.async_copy(rel_hbm.at[ri_v.at[sl]], r_rows.at[b], sems.at[b, 1])
    pltpu.async_copy(ent_hbm.at[ti_v.at[sl]], t_rows.at[b], sems.at[b, 2])


def _wait_chunk(ent_hbm, rel_hbm, hi_v, ri_v, ti_v, h_rows, r_rows, t_rows,
                sems, j, b):
    sl = pl.ds(j * C, C)
    pltpu.make_async_copy(ent_hbm.at[hi_v.at[sl]], h_rows.at[b],
                          sems.at[b, 0]).wait()
    pltpu.make_async_copy(rel_hbm.at[ri_v.at[sl]], r_rows.at[b],
                          sems.at[b, 1]).wait()
    pltpu.make_async_copy(ent_hbm.at[ti_v.at[sl]], t_rows.at[b],
                          sems.at[b, 2]).wait()


def _body(heads_hbm, rels_hbm, tails_hbm, ent_hbm, rel_hbm, out_hbm,
          hi_v, ri_v, ti_v, h_rows, r_rows, t_rows, part_v, out_v, sems):
    wid = lax.axis_index("s") * NC + lax.axis_index("c")
    base = wid * BPW

    ci = pltpu.async_copy(heads_hbm.at[pl.ds(base, BPW)], hi_v, sems.at[0, 0])
    cr = pltpu.async_copy(rels_hbm.at[pl.ds(base, BPW)], ri_v, sems.at[0, 1])
    ct = pltpu.async_copy(tails_hbm.at[pl.ds(base, BPW)], ti_v, sems.at[0, 2])
    ci.wait()
    cr.wait()
    ct.wait()

    lane = lax.iota(jnp.int32, L)
    lane_l = lane * L

    args = (ent_hbm, rel_hbm, hi_v, ri_v, ti_v, h_rows, r_rows, t_rows, sems)
    for b in range(NBUF):
        _start_chunk(*args, b, b)

    def compute_chunk(j, b):
        hb, rb, tb, pv = h_rows.at[b], r_rows.at[b], t_rows.at[b], part_v

        def row_body(r, carry):
            ps = [hb[r, pl.ds(k * L, L)] * rb[r, pl.ds(k * L, L)]
                  * tb[r, pl.ds(k * L, L)] for k in range(NSL)]
            while len(ps) > 1:
                ps = [ps[i] + ps[i + 1] for i in range(0, len(ps), 2)]
            pv[pl.ds(r * L, L)] = ps[0]
            return carry

        lax.fori_loop(0, C, row_body, 0, unroll=4)

        def group_body(g, carry):
            idx0 = g * (L * L) + lane_l
            acc = plsc.load_gather(part_v, [idx0])
            for k in range(1, L):
                acc += plsc.load_gather(part_v, [idx0 + k])
            out_v[pl.ds(j * C + g * L, L)] = acc
            return carry

        lax.fori_loop(0, C // L, group_body, 0)

    def outer_body(j, carry):
        b = lax.rem(j, NBUF)
        _wait_chunk(*args, j, b)
        compute_chunk(j, b)

        @pl.when(j < NCH - NBUF)
        def _():
            _start_chunk(*args, j + NBUF, b)
        return carry

    lax.fori_loop(0, NCH, outer_body, 0)

    pltpu.sync_copy(out_v, out_hbm.at[pl.ds(base, BPW)])


@jax.jit
def kernel(heads, relations, tails, entity_emb, relation_emb):
    f = pl.kernel(
        _body,
        out_type=jax.ShapeDtypeStruct((B,), jnp.float32),
        mesh=plsc.VectorSubcoreMesh(core_axis_name="c", subcore_axis_name="s",
                                    num_cores=NC, num_subcores=NS),
        compiler_params=pltpu.CompilerParams(needs_layout_passes=False),
        scratch_types=[
            pltpu.VMEM((BPW,), jnp.int32),
            pltpu.VMEM((BPW,), jnp.int32),
            pltpu.VMEM((BPW,), jnp.int32),
            pltpu.VMEM((NBUF, C, D), jnp.float32),
            pltpu.VMEM((NBUF, C, D), jnp.float32),
            pltpu.VMEM((NBUF, C, D), jnp.float32),
            pltpu.VMEM((C * L,), jnp.float32),
            pltpu.VMEM((BPW,), jnp.float32),
            pltpu.SemaphoreType.DMA((NBUF, 3)),
        ],
    )
    out = f(heads.astype(jnp.int32), relations.astype(jnp.int32),
            tails.astype(jnp.int32), entity_emb, relation_emb)
    return out.reshape(B, 1)
